# baseline (speedup 1.0000x reference)
import functools
import math

import jax
import jax.numpy as jnp
from jax import lax
from jax.experimental import pallas as pl
from jax.experimental.pallas import tpu as pltpu
from jax.experimental.pallas import tpu_sc as plsc

NUM_EXPERTS = 8
TOP_K = 2
CAPACITY_FACTOR = 1.25


def _gelu_exact(x):
    return x * 0.5 * (1.0 + lax.erf(x * 0.7071067811865476))


def _ffn_body(eob_ref, act_ref, x_ref, w1_ref, w2_ref, y_ref):
    fb = pl.program_id(1)

    @pl.when(act_ref[pl.program_id(0)] == 1)
    def _():
        h = jnp.dot(x_ref[...], w1_ref[0].astype(jnp.bfloat16),
                    preferred_element_type=jnp.float32)
        h = _gelu_exact(h)
        contrib = jnp.dot(h.astype(jnp.bfloat16), w2_ref[0].astype(jnp.bfloat16),
                          preferred_element_type=jnp.float32)

        @pl.when(fb == 0)
        def _():
            y_ref[...] = contrib

        @pl.when(fb != 0)
        def _():
            y_ref[...] += contrib


def _ffn_pallas(xg, w1, w2, eob, act, blk, fbs):
    npad, H = xg.shape
    E, _, F = w1.shape
    nblk = npad // blk
    nfb = F // fbs
    grid_spec = pltpu.PrefetchScalarGridSpec(
        num_scalar_prefetch=2,
        grid=(nblk, nfb),
        in_specs=[
            pl.BlockSpec((blk, H), lambda b, fb, e, a: (b, 0)),
            pl.BlockSpec((1, H, fbs), lambda b, fb, e, a: (e[b], 0, fb)),
            pl.BlockSpec((1, fbs, H), lambda b, fb, e, a: (e[b], fb, 0)),
        ],
        out_specs=pl.BlockSpec((blk, H), lambda b, fb, e, a: (b, 0)),
    )
    return pl.pallas_call(
        _ffn_body,
        grid_spec=grid_spec,
        out_shape=jax.ShapeDtypeStruct((npad, H), jnp.float32),
    )(eob, act, xg, w1, w2)


def _sc_combine(yg, p0t, p1t, wg0, wg1):
    N = p0t.shape[0]
    H = yg.shape[1]
    info = plsc.get_sparse_core_info()
    NC, NS, L = info.num_cores, info.num_subcores, info.num_lanes
    NW = NC * NS
    per_w = N // NW
    CH = 16
    nch = per_w // CH
    mesh = plsc.VectorSubcoreMesh(core_axis_name="c", subcore_axis_name="s")

    @functools.partial(
        pl.kernel,
        mesh=mesh,
        out_type=jax.ShapeDtypeStruct((N, H), jnp.float32),
        scratch_types=[
            pltpu.VMEM((per_w,), jnp.int32),
            pltpu.VMEM((per_w,), jnp.int32),
            pltpu.VMEM((per_w + 16,), jnp.float32),
            pltpu.VMEM((per_w + 16,), jnp.float32),
            pltpu.VMEM((2, CH, H), jnp.float32),
            pltpu.VMEM((2, CH, H), jnp.float32),
            pltpu.VMEM((CH, H), jnp.float32),
            pltpu.SemaphoreType.DMA,
            pltpu.SemaphoreType.DMA,
            pltpu.SemaphoreType.DMA,
            pltpu.SemaphoreType.DMA,
        ],
    )
    def k(yg_hbm, p0_hbm, p1_hbm, w0_hbm, w1_hbm, out_hbm,
          i0_v, i1_v, w0_v, w1_v, bufa, bufb, bufo, sa0, sa1, sb0, sb1):
        wid = lax.axis_index("s") * NC + lax.axis_index("c")
        base = wid * per_w
        pltpu.sync_copy(p0_hbm.at[pl.ds(base, per_w)], i0_v)
        pltpu.sync_copy(p1_hbm.at[pl.ds(base, per_w)], i1_v)
        pltpu.sync_copy(w0_hbm.at[pl.ds(base, per_w)], w0_v.at[pl.ds(0, per_w)])
        pltpu.sync_copy(w1_hbm.at[pl.ds(base, per_w)], w1_v.at[pl.ds(0, per_w)])
        sas = (sa0, sa1)
        sbs = (sb0, sb1)
        pltpu.async_copy(yg_hbm.at[i0_v.at[pl.ds(0, CH)]], bufa.at[0], sa0)
        pltpu.async_copy(yg_hbm.at[i1_v.at[pl.ds(0, CH)]], bufb.at[0], sb0)
        for c in range(nch):
            cur, nxt = c % 2, (c + 1) % 2
            pltpu.make_async_copy(yg_hbm.at[i0_v.at[pl.ds(0, CH)]],
                                  bufa.at[cur], sas[cur]).wait()
            pltpu.make_async_copy(yg_hbm.at[i1_v.at[pl.ds(0, CH)]],
                                  bufb.at[cur], sbs[cur]).wait()
            if c + 1 < nch:
                pltpu.async_copy(
                    yg_hbm.at[i0_v.at[pl.ds((c + 1) * CH, CH)]], bufa.at[nxt], sas[nxt])
                pltpu.async_copy(
                    yg_hbm.at[i1_v.at[pl.ds((c + 1) * CH, CH)]], bufb.at[nxt], sbs[nxt])

            def row_body(r, _):
                wa = jnp.full((L,), w0_v[pl.ds(c * CH + r, L)][0])
                wb = jnp.full((L,), w1_v[pl.ds(c * CH + r, L)][0])
                for j in range(H // L):
                    sl = pl.ds(j * L, L)
                    bufo[r, sl] = bufa[cur, r, sl] * wa + bufb[cur, r, sl] * wb
                return 0

            lax.fori_loop(0, CH, row_body, 0)
            pltpu.sync_copy(bufo, out_hbm.at[pl.ds(base + c * CH, CH)])

    return k(yg, p0t, p1t, wg0, wg1)


def _sc_dispatch(x_bf, src, npad):
    x32 = lax.bitcast_convert_type(
        x_bf.reshape(x_bf.shape[0], x_bf.shape[1] // 2, 2), jnp.int32)
    N, H = x32.shape
    info = plsc.get_sparse_core_info()
    NC, NS, L = info.num_cores, info.num_subcores, info.num_lanes
    NW = NC * NS
    per_w = npad // NW
    CH = 32
    nch = per_w // CH
    mesh = plsc.VectorSubcoreMesh(core_axis_name="c", subcore_axis_name="s")

    @functools.partial(
        pl.kernel,
        mesh=mesh,
        out_type=jax.ShapeDtypeStruct((npad, H), jnp.int32),
        scratch_types=[
            pltpu.VMEM((per_w,), jnp.int32),
            pltpu.VMEM((2, CH, H), jnp.int32),
            pltpu.SemaphoreType.DMA,
            pltpu.SemaphoreType.DMA,
        ],
    )
    def k(x_hbm, src_hbm, xg_hbm, idx_v, buf, sem0, sem1):
        wid = lax.axis_index("s") * NC + lax.axis_index("c")
        base = wid * per_w
        pltpu.sync_copy(src_hbm.at[pl.ds(base, per_w)], idx_v)
        sems = (sem0, sem1)
        pltpu.async_copy(x_hbm.at[idx_v.at[pl.ds(0, CH)]], buf.at[0], sem0)
        for c in range(nch):
            cur, nxt = c % 2, (c + 1) % 2
            pltpu.make_async_copy(x_hbm.at[idx_v.at[pl.ds(0, CH)]],
                                  buf.at[cur], sems[cur]).wait()
            if c + 1 < nch:
                pltpu.async_copy(x_hbm.at[idx_v.at[pl.ds((c + 1) * CH, CH)]],
                                 buf.at[nxt], sems[nxt])
            pltpu.sync_copy(buf.at[cur], xg_hbm.at[pl.ds(base + c * CH, CH)])

    xg32 = k(x32, src)
    return lax.bitcast_convert_type(xg32, jnp.bfloat16).reshape(npad, 2 * H)


def _route_and_pack(x_flat, W_router, blk):
    N, H = x_flat.shape
    E = NUM_EXPERTS
    cap = max(1, int(CAPACITY_FACTOR * N / E * TOP_K))

    logits = x_flat @ W_router.T
    probs = jax.nn.softmax(logits, axis=-1)
    e0 = jnp.argmax(probs, axis=-1)
    p0 = jnp.max(probs, axis=-1)
    masked = probs.at[jnp.arange(N), e0].set(-jnp.inf)
    e1 = jnp.argmax(masked, axis=-1)
    p1 = jnp.max(masked, axis=-1)
    wsum = p0 + p1
    w0 = p0 / wsum
    w1p = p1 / wsum

    oh0 = (e0[:, None] == jnp.arange(E)[None, :]).astype(jnp.int32)
    oh1 = (e1[:, None] == jnp.arange(E)[None, :]).astype(jnp.int32)
    rank0 = jnp.take_along_axis(jnp.cumsum(oh0, axis=0) - oh0, e0[:, None], 1)[:, 0]
    rank1 = jnp.take_along_axis(jnp.cumsum(oh1, axis=0) - oh1, e1[:, None], 1)[:, 0]
    cnt0 = jnp.minimum(jnp.sum(oh0, axis=0), cap)
    cnt1 = jnp.minimum(jnp.sum(oh1, axis=0), cap)
    n_e = cnt0 + cnt1
    mblk = (n_e + blk - 1) // blk
    baseblk = jnp.concatenate([jnp.zeros((1,), jnp.int32),
                               jnp.cumsum(mblk).astype(jnp.int32)])

    nblk = (N * TOP_K) // blk + E + 1
    npad = nblk * blk

    valid0 = rank0 < cap
    valid1 = rank1 < cap
    pos0 = baseblk[e0] * blk + rank0
    pos1 = baseblk[e1] * blk + cnt0[e1] + rank1

    toks = jnp.arange(N, dtype=jnp.int32)
    src = jnp.zeros((npad,), jnp.int32)
    src = src.at[jnp.where(valid0, pos0, npad)].set(toks, mode="drop")
    src = src.at[jnp.where(valid1, pos1, npad)].set(toks, mode="drop")

    p0t = jnp.where(valid0, pos0, 0).astype(jnp.int32)
    p1t = jnp.where(valid1, pos1, 0).astype(jnp.int32)
    wg0 = jnp.where(valid0, w0, 0.0).astype(jnp.float32)
    wg1 = jnp.where(valid1, w1p, 0.0).astype(jnp.float32)

    total_blk = baseblk[E]
    bidx = jnp.arange(nblk, dtype=jnp.int32)
    eob = jnp.minimum(
        jnp.sum(bidx[:, None] >= baseblk[None, 1:], axis=1), E - 1
    ).astype(jnp.int32)
    act = (bidx < total_blk).astype(jnp.int32)
    return src, p0t, p1t, wg0, wg1, eob, act, npad


def _moe(x, W_router, w1, w2, blk, fbs):
    B, T, H = x.shape
    x_flat = x.reshape(-1, H)
    src, p0t, p1t, wg0, wg1, eob, act, npad = _route_and_pack(x_flat, W_router, blk)
    xg = _sc_dispatch(x_flat.astype(jnp.bfloat16), src, npad)
    yg = _ffn_pallas(xg, w1, w2, eob, act, blk, fbs)
    out = _sc_combine(yg, p0t, p1t, wg0, wg1)
    return out.reshape(B, T, H)


def kernel(x, W_router, w1, w2):
    return _moe(x, W_router, w1, w2, blk=1024, fbs=1024)


# speedup vs baseline: 1.4338x; 1.4338x over previous
import functools
import math

import jax
import jax.numpy as jnp
from jax import lax
from jax.experimental import pallas as pl
from jax.experimental.pallas import tpu as pltpu
from jax.experimental.pallas import tpu_sc as plsc

NUM_EXPERTS = 8
TOP_K = 2
CAPACITY_FACTOR = 1.25


def _gelu_exact(x):
    return x * 0.5 * (1.0 + lax.erf(x * 0.7071067811865476))


def _ffn_body(eob_ref, act_ref, x_ref, w1_ref, w2_ref, y_ref):
    fb = pl.program_id(1)

    @pl.when(act_ref[pl.program_id(0)] == 1)
    def _():
        h = jnp.dot(x_ref[...].astype(jnp.bfloat16), w1_ref[0].astype(jnp.bfloat16),
                    preferred_element_type=jnp.float32)
        h = _gelu_exact(h)
        contrib = jnp.dot(h.astype(jnp.bfloat16), w2_ref[0].astype(jnp.bfloat16),
                          preferred_element_type=jnp.float32)

        @pl.when(fb == 0)
        def _():
            y_ref[...] = contrib

        @pl.when(fb != 0)
        def _():
            y_ref[...] += contrib


def _ffn_pallas(xg, w1, w2, eob, act, blk, fbs):
    npad, H = xg.shape
    E, _, F = w1.shape
    nblk = npad // blk
    nfb = F // fbs
    grid_spec = pltpu.PrefetchScalarGridSpec(
        num_scalar_prefetch=2,
        grid=(nblk, nfb),
        in_specs=[
            pl.BlockSpec((blk, H), lambda b, fb, e, a: (b, 0)),
            pl.BlockSpec((1, H, fbs), lambda b, fb, e, a: (e[b], 0, fb)),
            pl.BlockSpec((1, fbs, H), lambda b, fb, e, a: (e[b], fb, 0)),
        ],
        out_specs=pl.BlockSpec((blk, H), lambda b, fb, e, a: (b, 0)),
    )
    return pl.pallas_call(
        _ffn_body,
        grid_spec=grid_spec,
        out_shape=jax.ShapeDtypeStruct((npad, H), jnp.float32),
    )(eob, act, xg, w1, w2)


def _sc_combine(yg, p0t, p1t, wg0, wg1):
    N = p0t.shape[0]
    H = yg.shape[1]
    info = plsc.get_sparse_core_info()
    NC, NS, L = info.num_cores, info.num_subcores, info.num_lanes
    NW = NC * NS
    per_w = N // NW
    CH = 16
    nch = per_w // CH
    mesh = plsc.VectorSubcoreMesh(core_axis_name="c", subcore_axis_name="s")

    @functools.partial(
        pl.kernel,
        mesh=mesh,
        out_type=jax.ShapeDtypeStruct((N, H), jnp.float32),
        scratch_types=[
            pltpu.VMEM((per_w,), jnp.int32),
            pltpu.VMEM((per_w,), jnp.int32),
            pltpu.VMEM((per_w + 16,), jnp.float32),
            pltpu.VMEM((per_w + 16,), jnp.float32),
            pltpu.VMEM((2, CH, H), jnp.float32),
            pltpu.VMEM((2, CH, H), jnp.float32),
            pltpu.VMEM((CH, H), jnp.float32),
            pltpu.SemaphoreType.DMA,
            pltpu.SemaphoreType.DMA,
            pltpu.SemaphoreType.DMA,
            pltpu.SemaphoreType.DMA,
        ],
    )
    def k(yg_hbm, p0_hbm, p1_hbm, w0_hbm, w1_hbm, out_hbm,
          i0_v, i1_v, w0_v, w1_v, bufa, bufb, bufo, sa0, sa1, sb0, sb1):
        wid = lax.axis_index("s") * NC + lax.axis_index("c")
        base = wid * per_w
        pltpu.sync_copy(p0_hbm.at[pl.ds(base, per_w)], i0_v)
        pltpu.sync_copy(p1_hbm.at[pl.ds(base, per_w)], i1_v)
        pltpu.sync_copy(w0_hbm.at[pl.ds(base, per_w)], w0_v.at[pl.ds(0, per_w)])
        pltpu.sync_copy(w1_hbm.at[pl.ds(base, per_w)], w1_v.at[pl.ds(0, per_w)])
        sas = (sa0, sa1)
        sbs = (sb0, sb1)
        pltpu.async_copy(yg_hbm.at[i0_v.at[pl.ds(0, CH)]], bufa.at[0], sa0)
        pltpu.async_copy(yg_hbm.at[i1_v.at[pl.ds(0, CH)]], bufb.at[0], sb0)
        for c in range(nch):
            cur, nxt = c % 2, (c + 1) % 2
            pltpu.make_async_copy(yg_hbm.at[i0_v.at[pl.ds(0, CH)]],
                                  bufa.at[cur], sas[cur]).wait()
            pltpu.make_async_copy(yg_hbm.at[i1_v.at[pl.ds(0, CH)]],
                                  bufb.at[cur], sbs[cur]).wait()
            if c + 1 < nch:
                pltpu.async_copy(
                    yg_hbm.at[i0_v.at[pl.ds((c + 1) * CH, CH)]], bufa.at[nxt], sas[nxt])
                pltpu.async_copy(
                    yg_hbm.at[i1_v.at[pl.ds((c + 1) * CH, CH)]], bufb.at[nxt], sbs[nxt])

            def row_body(r, _):
                wa = jnp.full((L,), w0_v[pl.ds(c * CH + r, L)][0])
                wb = jnp.full((L,), w1_v[pl.ds(c * CH + r, L)][0])
                for j in range(H // L):
                    sl = pl.ds(j * L, L)
                    bufo[r, sl] = bufa[cur, r, sl] * wa + bufb[cur, r, sl] * wb
                return 0

            lax.fori_loop(0, CH, row_body, 0)
            pltpu.sync_copy(bufo, out_hbm.at[pl.ds(base + c * CH, CH)])

    return k(yg, p0t, p1t, wg0, wg1)


def _sc_dispatch(x_flat, src, npad):
    N, H = x_flat.shape
    info = plsc.get_sparse_core_info()
    NC, NS, L = info.num_cores, info.num_subcores, info.num_lanes
    NW = NC * NS
    per_w = npad // NW
    CH = 32
    nch = per_w // CH
    mesh = plsc.VectorSubcoreMesh(core_axis_name="c", subcore_axis_name="s")

    @functools.partial(
        pl.kernel,
        mesh=mesh,
        out_type=jax.ShapeDtypeStruct((npad, H), jnp.float32),
        scratch_types=[
            pltpu.VMEM((per_w,), jnp.int32),
            pltpu.VMEM((2, CH, H), jnp.float32),
            pltpu.SemaphoreType.DMA,
            pltpu.SemaphoreType.DMA,
        ],
    )
    def k(x_hbm, src_hbm, xg_hbm, idx_v, buf, sem0, sem1):
        wid = lax.axis_index("s") * NC + lax.axis_index("c")
        base = wid * per_w
        pltpu.sync_copy(src_hbm.at[pl.ds(base, per_w)], idx_v)
        sems = (sem0, sem1)
        pltpu.async_copy(x_hbm.at[idx_v.at[pl.ds(0, CH)]], buf.at[0], sem0)
        for c in range(nch):
            cur, nxt = c % 2, (c + 1) % 2
            pltpu.make_async_copy(x_hbm.at[idx_v.at[pl.ds(0, CH)]],
                                  buf.at[cur], sems[cur]).wait()
            if c + 1 < nch:
                pltpu.async_copy(x_hbm.at[idx_v.at[pl.ds((c + 1) * CH, CH)]],
                                 buf.at[nxt], sems[nxt])
            pltpu.sync_copy(buf.at[cur], xg_hbm.at[pl.ds(base + c * CH, CH)])

    return k(x_flat, src)


def _route_and_pack(x_flat, W_router, blk):
    N, H = x_flat.shape
    E = NUM_EXPERTS
    cap = max(1, int(CAPACITY_FACTOR * N / E * TOP_K))

    logits = x_flat @ W_router.T
    probs = jax.nn.softmax(logits, axis=-1)
    e0 = jnp.argmax(probs, axis=-1)
    p0 = jnp.max(probs, axis=-1)
    masked = probs.at[jnp.arange(N), e0].set(-jnp.inf)
    e1 = jnp.argmax(masked, axis=-1)
    p1 = jnp.max(masked, axis=-1)
    wsum = p0 + p1
    w0 = p0 / wsum
    w1p = p1 / wsum

    oh0 = (e0[:, None] == jnp.arange(E)[None, :]).astype(jnp.int32)
    oh1 = (e1[:, None] == jnp.arange(E)[None, :]).astype(jnp.int32)
    rank0 = jnp.take_along_axis(jnp.cumsum(oh0, axis=0) - oh0, e0[:, None], 1)[:, 0]
    rank1 = jnp.take_along_axis(jnp.cumsum(oh1, axis=0) - oh1, e1[:, None], 1)[:, 0]
    cnt0 = jnp.minimum(jnp.sum(oh0, axis=0), cap)
    cnt1 = jnp.minimum(jnp.sum(oh1, axis=0), cap)
    n_e = cnt0 + cnt1
    mblk = (n_e + blk - 1) // blk
    baseblk = jnp.concatenate([jnp.zeros((1,), jnp.int32),
                               jnp.cumsum(mblk).astype(jnp.int32)])

    nblk = (N * TOP_K) // blk + E + 1
    npad = nblk * blk

    valid0 = rank0 < cap
    valid1 = rank1 < cap
    pos0 = baseblk[e0] * blk + rank0
    pos1 = baseblk[e1] * blk + cnt0[e1] + rank1

    toks = jnp.arange(N, dtype=jnp.int32)
    src = jnp.zeros((npad,), jnp.int32)
    src = src.at[jnp.where(valid0, pos0, npad)].set(toks, mode="drop")
    src = src.at[jnp.where(valid1, pos1, npad)].set(toks, mode="drop")

    p0t = jnp.where(valid0, pos0, 0).astype(jnp.int32)
    p1t = jnp.where(valid1, pos1, 0).astype(jnp.int32)
    wg0 = jnp.where(valid0, w0, 0.0).astype(jnp.float32)
    wg1 = jnp.where(valid1, w1p, 0.0).astype(jnp.float32)

    total_blk = baseblk[E]
    bidx = jnp.arange(nblk, dtype=jnp.int32)
    eob = jnp.minimum(
        jnp.sum(bidx[:, None] >= baseblk[None, 1:], axis=1), E - 1
    ).astype(jnp.int32)
    act = (bidx < total_blk).astype(jnp.int32)
    return src, p0t, p1t, wg0, wg1, eob, act, npad


def _moe(x, W_router, w1, w2, blk, fbs):
    B, T, H = x.shape
    x_flat = x.reshape(-1, H)
    src, p0t, p1t, wg0, wg1, eob, act, npad = _route_and_pack(x_flat, W_router, blk)
    xg = _sc_dispatch(x_flat, src, npad)
    yg = _ffn_pallas(xg, w1, w2, eob, act, blk, fbs)
    out = _sc_combine(yg, p0t, p1t, wg0, wg1)
    return out.reshape(B, T, H)


def kernel(x, W_router, w1, w2):
    return _moe(x, W_router, w1, w2, blk=1024, fbs=1024)


# speedup vs baseline: 2.1080x; 1.4702x over previous
import functools
import math

import jax
import jax.numpy as jnp
from jax import lax
from jax.experimental import pallas as pl
from jax.experimental.pallas import tpu as pltpu
from jax.experimental.pallas import tpu_sc as plsc

NUM_EXPERTS = 8
TOP_K = 2
CAPACITY_FACTOR = 1.25


def _gelu_exact(x):
    return x * 0.5 * (1.0 + lax.erf(x * 0.7071067811865476))


def _ffn_body(eob_ref, act_ref, x_ref, w1_ref, w2_ref, y_ref):
    fb = pl.program_id(1)

    @pl.when(act_ref[pl.program_id(0)] == 1)
    def _():
        h = jnp.dot(x_ref[...], w1_ref[0].astype(jnp.bfloat16),
                    preferred_element_type=jnp.float32)
        h = _gelu_exact(h)
        contrib = jnp.dot(h.astype(jnp.bfloat16), w2_ref[0].astype(jnp.bfloat16),
                          preferred_element_type=jnp.float32)

        @pl.when(fb == 0)
        def _():
            y_ref[...] = contrib

        @pl.when(fb != 0)
        def _():
            y_ref[...] += contrib


def _ffn_pallas(xg, w1, w2, eob, act, blk, fbs):
    npad, H = xg.shape
    E, _, F = w1.shape
    nblk = npad // blk
    nfb = F // fbs
    grid_spec = pltpu.PrefetchScalarGridSpec(
        num_scalar_prefetch=2,
        grid=(nblk, nfb),
        in_specs=[
            pl.BlockSpec((blk, H), lambda b, fb, e, a: (b, 0)),
            pl.BlockSpec((1, H, fbs), lambda b, fb, e, a: (e[b], 0, fb)),
            pl.BlockSpec((1, fbs, H), lambda b, fb, e, a: (e[b], fb, 0)),
        ],
        out_specs=pl.BlockSpec((blk, H), lambda b, fb, e, a: (b, 0)),
    )
    return pl.pallas_call(
        _ffn_body,
        grid_spec=grid_spec,
        out_shape=jax.ShapeDtypeStruct((npad, H), jnp.float32),
    )(eob, act, xg, w1, w2)


def _sc_combine(yg, p0t, p1t, wg0, wg1):
    N = p0t.shape[0]
    H = yg.shape[1]
    info = plsc.get_sparse_core_info()
    NC, NS, L = info.num_cores, info.num_subcores, info.num_lanes
    NW = NC * NS
    per_w = N // NW
    CH = 16
    nch = per_w // CH
    mesh = plsc.VectorSubcoreMesh(core_axis_name="c", subcore_axis_name="s")

    @functools.partial(
        pl.kernel,
        mesh=mesh,
        out_type=jax.ShapeDtypeStruct((N, H), jnp.float32),
        scratch_types=[
            pltpu.VMEM((per_w,), jnp.int32),
            pltpu.VMEM((per_w,), jnp.int32),
            pltpu.VMEM((per_w + 16,), jnp.float32),
            pltpu.VMEM((per_w + 16,), jnp.float32),
            pltpu.VMEM((2, CH, H), jnp.float32),
            pltpu.VMEM((2, CH, H), jnp.float32),
            pltpu.VMEM((CH, H), jnp.float32),
            pltpu.SemaphoreType.DMA,
            pltpu.SemaphoreType.DMA,
            pltpu.SemaphoreType.DMA,
            pltpu.SemaphoreType.DMA,
        ],
    )
    def k(yg_hbm, p0_hbm, p1_hbm, w0_hbm, w1_hbm, out_hbm,
          i0_v, i1_v, w0_v, w1_v, bufa, bufb, bufo, sa0, sa1, sb0, sb1):
        wid = lax.axis_index("s") * NC + lax.axis_index("c")
        base = wid * per_w
        pltpu.sync_copy(p0_hbm.at[pl.ds(base, per_w)], i0_v)
        pltpu.sync_copy(p1_hbm.at[pl.ds(base, per_w)], i1_v)
        pltpu.sync_copy(w0_hbm.at[pl.ds(base, per_w)], w0_v.at[pl.ds(0, per_w)])
        pltpu.sync_copy(w1_hbm.at[pl.ds(base, per_w)], w1_v.at[pl.ds(0, per_w)])
        sas = (sa0, sa1)
        sbs = (sb0, sb1)
        pltpu.async_copy(yg_hbm.at[i0_v.at[pl.ds(0, CH)]], bufa.at[0], sa0)
        pltpu.async_copy(yg_hbm.at[i1_v.at[pl.ds(0, CH)]], bufb.at[0], sb0)
        for c in range(nch):
            cur, nxt = c % 2, (c + 1) % 2
            pltpu.make_async_copy(yg_hbm.at[i0_v.at[pl.ds(0, CH)]],
                                  bufa.at[cur], sas[cur]).wait()
            pltpu.make_async_copy(yg_hbm.at[i1_v.at[pl.ds(0, CH)]],
                                  bufb.at[cur], sbs[cur]).wait()
            if c + 1 < nch:
                pltpu.async_copy(
                    yg_hbm.at[i0_v.at[pl.ds((c + 1) * CH, CH)]], bufa.at[nxt], sas[nxt])
                pltpu.async_copy(
                    yg_hbm.at[i1_v.at[pl.ds((c + 1) * CH, CH)]], bufb.at[nxt], sbs[nxt])

            def row_body(r, _):
                wa = jnp.full((L,), w0_v[pl.ds(c * CH + r, L)][0])
                wb = jnp.full((L,), w1_v[pl.ds(c * CH + r, L)][0])
                for j in range(H // L):
                    sl = pl.ds(j * L, L)
                    bufo[r, sl] = bufa[cur, r, sl] * wa + bufb[cur, r, sl] * wb
                return 0

            lax.fori_loop(0, CH, row_body, 0)
            pltpu.sync_copy(bufo, out_hbm.at[pl.ds(base + c * CH, CH)])

    return k(yg, p0t, p1t, wg0, wg1)


def _sc_dispatch(x_flat, src, npad):
    N, H = x_flat.shape
    info = plsc.get_sparse_core_info()
    NC, NS, L = info.num_cores, info.num_subcores, info.num_lanes
    NW = NC * NS
    per_w = npad // NW
    CH = 32
    nch = per_w // CH
    mesh = plsc.VectorSubcoreMesh(core_axis_name="c", subcore_axis_name="s")

    @functools.partial(
        pl.kernel,
        mesh=mesh,
        out_type=jax.ShapeDtypeStruct((npad, H), jnp.float32),
        scratch_types=[
            pltpu.VMEM((per_w,), jnp.int32),
            pltpu.VMEM((2, CH, H), jnp.float32),
            pltpu.SemaphoreType.DMA,
            pltpu.SemaphoreType.DMA,
        ],
    )
    def k(x_hbm, src_hbm, xg_hbm, idx_v, buf, sem0, sem1):
        wid = lax.axis_index("s") * NC + lax.axis_index("c")
        base = wid * per_w
        pltpu.sync_copy(src_hbm.at[pl.ds(base, per_w)], idx_v)
        sems = (sem0, sem1)
        pltpu.async_copy(x_hbm.at[idx_v.at[pl.ds(0, CH)]], buf.at[0], sem0)
        for c in range(nch):
            cur, nxt = c % 2, (c + 1) % 2
            pltpu.make_async_copy(x_hbm.at[idx_v.at[pl.ds(0, CH)]],
                                  buf.at[cur], sems[cur]).wait()
            if c + 1 < nch:
                pltpu.async_copy(x_hbm.at[idx_v.at[pl.ds((c + 1) * CH, CH)]],
                                 buf.at[nxt], sems[nxt])
            pltpu.sync_copy(buf.at[cur], xg_hbm.at[pl.ds(base + c * CH, CH)])

    return k(x_flat, src)


def _route_and_pack(x_flat, W_router, blk):
    N, H = x_flat.shape
    E = NUM_EXPERTS
    cap = max(1, int(CAPACITY_FACTOR * N / E * TOP_K))

    logits = x_flat @ W_router.T
    probs = jax.nn.softmax(logits, axis=-1)
    e0 = jnp.argmax(probs, axis=-1)
    p0 = jnp.max(probs, axis=-1)
    masked = probs.at[jnp.arange(N), e0].set(-jnp.inf)
    e1 = jnp.argmax(masked, axis=-1)
    p1 = jnp.max(masked, axis=-1)
    wsum = p0 + p1
    w0 = p0 / wsum
    w1p = p1 / wsum

    oh0 = (e0[:, None] == jnp.arange(E)[None, :]).astype(jnp.int32)
    oh1 = (e1[:, None] == jnp.arange(E)[None, :]).astype(jnp.int32)
    rank0 = jnp.take_along_axis(jnp.cumsum(oh0, axis=0) - oh0, e0[:, None], 1)[:, 0]
    rank1 = jnp.take_along_axis(jnp.cumsum(oh1, axis=0) - oh1, e1[:, None], 1)[:, 0]
    cnt0 = jnp.minimum(jnp.sum(oh0, axis=0), cap)
    cnt1 = jnp.minimum(jnp.sum(oh1, axis=0), cap)
    n_e = cnt0 + cnt1
    mblk = (n_e + blk - 1) // blk
    baseblk = jnp.concatenate([jnp.zeros((1,), jnp.int32),
                               jnp.cumsum(mblk).astype(jnp.int32)])

    nblk = (N * TOP_K) // blk + E + 1
    npad = nblk * blk

    valid0 = rank0 < cap
    valid1 = rank1 < cap
    pos0 = baseblk[e0] * blk + rank0
    pos1 = baseblk[e1] * blk + cnt0[e1] + rank1

    toks = jnp.arange(N, dtype=jnp.int32)
    src = jnp.zeros((npad,), jnp.int32)
    src = src.at[jnp.where(valid0, pos0, npad)].set(toks, mode="drop")
    src = src.at[jnp.where(valid1, pos1, npad)].set(toks, mode="drop")

    p0t = jnp.where(valid0, pos0, 0).astype(jnp.int32)
    p1t = jnp.where(valid1, pos1, 0).astype(jnp.int32)
    wg0 = jnp.where(valid0, w0, 0.0).astype(jnp.float32)
    wg1 = jnp.where(valid1, w1p, 0.0).astype(jnp.float32)

    total_blk = baseblk[E]
    bidx = jnp.arange(nblk, dtype=jnp.int32)
    eob = jnp.minimum(
        jnp.sum(bidx[:, None] >= baseblk[None, 1:], axis=1), E - 1
    ).astype(jnp.int32)
    act = (bidx < total_blk).astype(jnp.int32)
    return src, p0t, p1t, wg0, wg1, eob, act, npad


def _moe(x, W_router, w1, w2, blk, fbs):
    B, T, H = x.shape
    x_flat = x.reshape(-1, H)
    src, p0t, p1t, wg0, wg1, eob, act, npad = _route_and_pack(x_flat, W_router, blk)
    xg = x_flat.astype(jnp.bfloat16)[src]
    yg = _ffn_pallas(xg, w1, w2, eob, act, blk, fbs)
    out = _sc_combine(yg, p0t, p1t, wg0, wg1)
    return out.reshape(B, T, H)


def kernel(x, W_router, w1, w2):
    return _moe(x, W_router, w1, w2, blk=1024, fbs=1024)


# speedup vs baseline: 2.1975x; 1.0425x over previous
import functools
import math

import jax
import jax.numpy as jnp
from jax import lax
from jax.experimental import pallas as pl
from jax.experimental.pallas import tpu as pltpu
from jax.experimental.pallas import tpu_sc as plsc

NUM_EXPERTS = 8
TOP_K = 2
CAPACITY_FACTOR = 1.25


def _gelu_exact(x):
    return x * 0.5 * (1.0 + lax.erf(x * 0.7071067811865476))


def _ffn_body(eob_ref, act_ref, x_ref, w1_ref, w2_ref, y_ref):
    fb = pl.program_id(1)

    @pl.when(act_ref[pl.program_id(0)] == 1)
    def _():
        h = jnp.dot(x_ref[...], w1_ref[0].astype(jnp.bfloat16),
                    preferred_element_type=jnp.float32)
        h = _gelu_exact(h)
        contrib = jnp.dot(h.astype(jnp.bfloat16), w2_ref[0].astype(jnp.bfloat16),
                          preferred_element_type=jnp.float32)

        @pl.when(fb == 0)
        def _():
            y_ref[...] = contrib

        @pl.when(fb != 0)
        def _():
            y_ref[...] += contrib


def _ffn_pallas(xg, w1, w2, eob, act, blk, fbs):
    npad, H = xg.shape
    E, _, F = w1.shape
    nblk = npad // blk
    nfb = F // fbs
    grid_spec = pltpu.PrefetchScalarGridSpec(
        num_scalar_prefetch=2,
        grid=(nblk, nfb),
        in_specs=[
            pl.BlockSpec((blk, H), lambda b, fb, e, a: (b, 0)),
            pl.BlockSpec((1, H, fbs), lambda b, fb, e, a: (e[b], 0, fb)),
            pl.BlockSpec((1, fbs, H), lambda b, fb, e, a: (e[b], fb, 0)),
        ],
        out_specs=pl.BlockSpec((blk, H), lambda b, fb, e, a: (b, 0)),
    )
    return pl.pallas_call(
        _ffn_body,
        grid_spec=grid_spec,
        out_shape=jax.ShapeDtypeStruct((npad, H), jnp.float32),
    )(eob, act, xg, w1, w2)


def _sc_combine(yg, p0t, p1t, wg0, wg1):
    N = p0t.shape[0]
    H = yg.shape[1]
    info = plsc.get_sparse_core_info()
    NC, NS, L = info.num_cores, info.num_subcores, info.num_lanes
    NW = NC * NS
    per_w = N // NW
    CH = 16
    nch = per_w // CH
    mesh = plsc.VectorSubcoreMesh(core_axis_name="c", subcore_axis_name="s")

    @functools.partial(
        pl.kernel,
        mesh=mesh,
        out_type=jax.ShapeDtypeStruct((N, H), jnp.float32),
        scratch_types=[
            pltpu.VMEM((per_w,), jnp.int32),
            pltpu.VMEM((per_w,), jnp.int32),
            pltpu.VMEM((per_w + 16,), jnp.float32),
            pltpu.VMEM((per_w + 16,), jnp.float32),
            pltpu.VMEM((2, CH, H), jnp.float32),
            pltpu.VMEM((2, CH, H), jnp.float32),
            pltpu.VMEM((CH, H), jnp.float32),
            pltpu.SemaphoreType.DMA,
            pltpu.SemaphoreType.DMA,
            pltpu.SemaphoreType.DMA,
            pltpu.SemaphoreType.DMA,
        ],
    )
    def k(yg_hbm, p0_hbm, p1_hbm, w0_hbm, w1_hbm, out_hbm,
          i0_v, i1_v, w0_v, w1_v, bufa, bufb, bufo, sa0, sa1, sb0, sb1):
        wid = lax.axis_index("s") * NC + lax.axis_index("c")
        base = wid * per_w
        pltpu.sync_copy(p0_hbm.at[pl.ds(base, per_w)], i0_v)
        pltpu.sync_copy(p1_hbm.at[pl.ds(base, per_w)], i1_v)
        pltpu.sync_copy(w0_hbm.at[pl.ds(base, per_w)], w0_v.at[pl.ds(0, per_w)])
        pltpu.sync_copy(w1_hbm.at[pl.ds(base, per_w)], w1_v.at[pl.ds(0, per_w)])
        sas = (sa0, sa1)
        sbs = (sb0, sb1)
        pltpu.async_copy(yg_hbm.at[i0_v.at[pl.ds(0, CH)]], bufa.at[0], sa0)
        pltpu.async_copy(yg_hbm.at[i1_v.at[pl.ds(0, CH)]], bufb.at[0], sb0)
        for c in range(nch):
            cur, nxt = c % 2, (c + 1) % 2
            pltpu.make_async_copy(yg_hbm.at[i0_v.at[pl.ds(0, CH)]],
                                  bufa.at[cur], sas[cur]).wait()
            pltpu.make_async_copy(yg_hbm.at[i1_v.at[pl.ds(0, CH)]],
                                  bufb.at[cur], sbs[cur]).wait()
            if c + 1 < nch:
                pltpu.async_copy(
                    yg_hbm.at[i0_v.at[pl.ds((c + 1) * CH, CH)]], bufa.at[nxt], sas[nxt])
                pltpu.async_copy(
                    yg_hbm.at[i1_v.at[pl.ds((c + 1) * CH, CH)]], bufb.at[nxt], sbs[nxt])

            def row_body(r, _):
                wa = jnp.full((L,), w0_v[pl.ds(c * CH + r, L)][0])
                wb = jnp.full((L,), w1_v[pl.ds(c * CH + r, L)][0])
                for j in range(H // L):
                    sl = pl.ds(j * L, L)
                    bufo[r, sl] = bufa[cur, r, sl] * wa + bufb[cur, r, sl] * wb
                return 0

            lax.fori_loop(0, CH, row_body, 0)
            pltpu.sync_copy(bufo, out_hbm.at[pl.ds(base + c * CH, CH)])

    return k(yg, p0t, p1t, wg0, wg1)


def _sc_dispatch(x_flat, src, npad):
    N, H = x_flat.shape
    info = plsc.get_sparse_core_info()
    NC, NS, L = info.num_cores, info.num_subcores, info.num_lanes
    NW = NC * NS
    per_w = npad // NW
    CH = 32
    nch = per_w // CH
    mesh = plsc.VectorSubcoreMesh(core_axis_name="c", subcore_axis_name="s")

    @functools.partial(
        pl.kernel,
        mesh=mesh,
        out_type=jax.ShapeDtypeStruct((npad, H), jnp.float32),
        scratch_types=[
            pltpu.VMEM((per_w,), jnp.int32),
            pltpu.VMEM((2, CH, H), jnp.float32),
            pltpu.SemaphoreType.DMA,
            pltpu.SemaphoreType.DMA,
        ],
    )
    def k(x_hbm, src_hbm, xg_hbm, idx_v, buf, sem0, sem1):
        wid = lax.axis_index("s") * NC + lax.axis_index("c")
        base = wid * per_w
        pltpu.sync_copy(src_hbm.at[pl.ds(base, per_w)], idx_v)
        sems = (sem0, sem1)
        pltpu.async_copy(x_hbm.at[idx_v.at[pl.ds(0, CH)]], buf.at[0], sem0)
        for c in range(nch):
            cur, nxt = c % 2, (c + 1) % 2
            pltpu.make_async_copy(x_hbm.at[idx_v.at[pl.ds(0, CH)]],
                                  buf.at[cur], sems[cur]).wait()
            if c + 1 < nch:
                pltpu.async_copy(x_hbm.at[idx_v.at[pl.ds((c + 1) * CH, CH)]],
                                 buf.at[nxt], sems[nxt])
            pltpu.sync_copy(buf.at[cur], xg_hbm.at[pl.ds(base + c * CH, CH)])

    return k(x_flat, src)


def _route_and_pack(x_flat, W_router, blk):
    N, H = x_flat.shape
    E = NUM_EXPERTS
    cap = max(1, int(CAPACITY_FACTOR * N / E * TOP_K))

    logits = x_flat @ W_router.T
    probs = jax.nn.softmax(logits, axis=-1)
    e0 = jnp.argmax(probs, axis=-1)
    p0 = jnp.max(probs, axis=-1)
    masked = probs.at[jnp.arange(N), e0].set(-jnp.inf)
    e1 = jnp.argmax(masked, axis=-1)
    p1 = jnp.max(masked, axis=-1)
    wsum = p0 + p1
    w0 = p0 / wsum
    w1p = p1 / wsum

    oh0 = (e0[:, None] == jnp.arange(E)[None, :]).astype(jnp.int32)
    oh1 = (e1[:, None] == jnp.arange(E)[None, :]).astype(jnp.int32)
    rank0 = jnp.take_along_axis(jnp.cumsum(oh0, axis=0) - oh0, e0[:, None], 1)[:, 0]
    rank1 = jnp.take_along_axis(jnp.cumsum(oh1, axis=0) - oh1, e1[:, None], 1)[:, 0]
    cnt0 = jnp.minimum(jnp.sum(oh0, axis=0), cap)
    cnt1 = jnp.minimum(jnp.sum(oh1, axis=0), cap)
    n_e = cnt0 + cnt1
    mblk = (n_e + blk - 1) // blk
    baseblk = jnp.concatenate([jnp.zeros((1,), jnp.int32),
                               jnp.cumsum(mblk).astype(jnp.int32)])

    nblk = (N * TOP_K) // blk + E + 1
    npad = nblk * blk

    valid0 = rank0 < cap
    valid1 = rank1 < cap
    pos0 = baseblk[e0] * blk + rank0
    pos1 = baseblk[e1] * blk + cnt0[e1] + rank1

    toks = jnp.arange(N, dtype=jnp.int32)
    src = jnp.zeros((npad,), jnp.int32)
    src = src.at[jnp.where(valid0, pos0, npad)].set(toks, mode="drop")
    src = src.at[jnp.where(valid1, pos1, npad)].set(toks, mode="drop")

    p0t = jnp.where(valid0, pos0, 0).astype(jnp.int32)
    p1t = jnp.where(valid1, pos1, 0).astype(jnp.int32)
    wg0 = jnp.where(valid0, w0, 0.0).astype(jnp.float32)
    wg1 = jnp.where(valid1, w1p, 0.0).astype(jnp.float32)

    total_blk = baseblk[E]
    bidx = jnp.arange(nblk, dtype=jnp.int32)
    eob = jnp.minimum(
        jnp.sum(bidx[:, None] >= baseblk[None, 1:], axis=1), E - 1
    ).astype(jnp.int32)
    act = (bidx < total_blk).astype(jnp.int32)
    return src, p0t, p1t, wg0, wg1, eob, act, npad


def _moe(x, W_router, w1, w2, blk, fbs):
    B, T, H = x.shape
    x_flat = x.reshape(-1, H)
    src, p0t, p1t, wg0, wg1, eob, act, npad = _route_and_pack(x_flat, W_router, blk)
    xg = x_flat.astype(jnp.bfloat16)[src]
    yg = _ffn_pallas(xg, w1, w2, eob, act, blk, fbs)
    out = _sc_combine(yg, p0t, p1t, wg0, wg1)
    return out.reshape(B, T, H)


def kernel(x, W_router, w1, w2):
    return _moe(x, W_router, w1, w2, blk=1024, fbs=2048)
